# baseline (speedup 1.0000x reference)
import functools

import jax
import jax.numpy as jnp
from jax import lax
from jax.experimental import pallas as pl
from jax.experimental.pallas import tpu as pltpu
from jax.experimental.pallas import tpu_sc as plsc

B, T, D = 16, 2048, 256
VOCAB = 100000

_NC = 2
_NS = 16
_LANES = 16
_NW = _NC * _NS
_N_IDX = B * T
_CHUNK = _N_IDX // _NW


_SUB = 8
_SUBW = _CHUNK // _SUB


def _sc_gather(weights, idx3):
    mesh = plsc.VectorSubcoreMesh(core_axis_name="c", subcore_axis_name="s")
    nw, sub, subw = idx3.shape

    @functools.partial(
        pl.kernel,
        mesh=mesh,
        out_type=jax.ShapeDtypeStruct((nw, sub, subw), jnp.float32),
        scratch_types=[
            pltpu.VMEM((sub, subw), jnp.int32),
            pltpu.VMEM((sub, subw), jnp.float32),
            pltpu.SemaphoreType.DMA,
        ],
        compiler_params=pltpu.CompilerParams(needs_layout_passes=False),
    )
    def gather_kernel(w_hbm, idx_hbm, out_hbm, idx_v, rows_v, sem):
        wid = lax.axis_index("s") * _NC + lax.axis_index("c")
        pltpu.sync_copy(idx_hbm.at[wid], idx_v)
        copies = [
            pltpu.async_copy(w_hbm.at[idx_v.at[j]], rows_v.at[j], sem)
            for j in range(sub)
        ]
        for c in copies:
            c.wait()
        pltpu.sync_copy(rows_v, out_hbm.at[wid])

    return gather_kernel(weights, idx3)


_NROW = 4
_GB = B // _NROW


def _one_row(length, vs, w_row_raw, y_ref, yh_ref):
    pos = lax.broadcasted_iota(jnp.int32, (1, T), 1)
    maskf = (pos < length).astype(jnp.float32)
    w_row = w_row_raw * maskf
    a = jnp.concatenate([maskf, w_row], axis=0)
    acc = jnp.dot(a, vs, preferred_element_type=jnp.float32)
    s = acc[0:1, :]
    denom = jnp.sqrt(jnp.sum(jnp.abs(s)))
    y_ref[0, :, :] = s / denom
    yh_ref[0, :, :] = acc[1:2, :]


def _tc_body(len_ref, *refs, base, gb):
    vs_refs = refs[:_NROW]
    w_refs = refs[_NROW:2 * _NROW]
    y_refs = refs[2 * _NROW:3 * _NROW]
    yh_refs = refs[3 * _NROW:]
    b = pl.program_id(0)
    for k in range(_NROW):
        _one_row(len_ref[base + b + k * gb], vs_refs[k][0], w_refs[k][0],
                 y_refs[k], yh_refs[k])


def _tc_half(lens, vector_sequence, w3, half):
    base = half * (B // 2)

    def _off(k, b0):
        return lambda b: (b0 + b + k * 2, 0, 0)

    vs_specs = [pl.BlockSpec((1, T, D), _off(k, base)) for k in range(_NROW)]
    w_specs = [pl.BlockSpec((1, 1, T), _off(k, 0)) for k in range(_NROW)]
    out_spec = pl.BlockSpec((1, 1, D), lambda b: (b, 0, 0))
    out_ty = jax.ShapeDtypeStruct((2, 1, D), jnp.float32)
    return pl.pallas_call(
        functools.partial(_tc_body, base=base, gb=2),
        grid=(2,),
        in_specs=[
            pl.BlockSpec(memory_space=pltpu.SMEM),
            *vs_specs,
            *w_specs,
        ],
        out_specs=[out_spec] * (2 * _NROW),
        out_shape=[out_ty] * (2 * _NROW),
    )(lens, *([vector_sequence] * _NROW), *([w3] * _NROW))


def kernel(vector_sequence, sentence_length, word_sequence, weights):
    idx = word_sequence.astype(jnp.int32)
    lens = sentence_length.astype(jnp.int32)
    halves = []
    w3s = []
    for half in range(2):
        idx3 = idx[half * 8:(half + 1) * 8].reshape(_NW, _SUB // 2, _SUBW)
        w3s.append(_sc_gather(weights, idx3).reshape(8, 1, T))
    for half in range(2):
        halves.append(_tc_half(lens, vector_sequence, w3s[half], half))
    y = jnp.concatenate(
        [jnp.concatenate(h[:_NROW], axis=0) for h in halves], axis=0
    ).reshape(B, D)
    y_hat = jnp.concatenate(
        [jnp.concatenate(h[_NROW:], axis=0) for h in halves], axis=0
    ).reshape(B, D)
    return y, y_hat
